# baseline (speedup 1.0000x reference)
import functools

import jax
import jax.numpy as jnp
from jax import lax
from jax.experimental import pallas as pl
from jax.experimental.pallas import tpu as pltpu


def _support_body(x_ref, w_ref, out_ref):
    out_ref[...] = lax.dot_general(
        x_ref[...], w_ref[...], (((1,), (0,)), ((), ())),
        preferred_element_type=jnp.float32)


def _agg_body(nk, bk, k_valid, adj_ref, sup_ref, b_ref, out_ref):
    k = pl.program_id(1)

    @pl.when(k == 0)
    def _init():
        out_ref[...] = jnp.zeros_like(out_ref)

    a = adj_ref[...]
    col = k * bk + lax.broadcasted_iota(jnp.int32, a.shape, 1)
    a = jnp.where(col < k_valid, a, 0.0)
    sup = sup_ref[pl.ds(k * bk, bk), :]
    out_ref[...] += lax.dot_general(
        a, sup, (((1,), (0,)), ((), ())),
        preferred_element_type=jnp.float32)

    @pl.when(k == nk - 1)
    def _finish():
        out_ref[...] = jnp.maximum(out_ref[...] + b_ref[...], 0.0)


@jax.jit
def kernel(x, adj, W, b):
    M, K = adj.shape
    D_in = x.shape[1]
    D_out = W.shape[1]

    BK = min(2048, pl.cdiv(K, 128) * 128)
    nk = pl.cdiv(K, BK)
    KP = nk * BK

    x_pad = jnp.pad(x, ((0, KP - K), (0, 0))) if KP != K else x

    support = pl.pallas_call(
        _support_body,
        grid=(KP // BK,),
        in_specs=[
            pl.BlockSpec((BK, D_in), lambda i: (i, 0)),
            pl.BlockSpec((D_in, D_out), lambda i: (0, 0)),
        ],
        out_specs=pl.BlockSpec((BK, D_out), lambda i: (i, 0)),
        out_shape=jax.ShapeDtypeStruct((KP, D_out), jnp.float32),
    )(x_pad, W)

    BM = 400 if M % 400 == 0 else min(M, 256)
    nm = pl.cdiv(M, BM)

    out = pl.pallas_call(
        functools.partial(_agg_body, nk, BK, K),
        grid=(nm, nk),
        in_specs=[
            pl.BlockSpec((BM, BK), lambda i, k: (i, k)),
            pl.BlockSpec((KP, D_out), lambda i, k: (0, 0)),
            pl.BlockSpec((1, D_out), lambda i, k: (0, 0)),
        ],
        out_specs=pl.BlockSpec((BM, D_out), lambda i, k: (i, 0)),
        out_shape=jax.ShapeDtypeStruct((M, D_out), jnp.float32),
        compiler_params=pltpu.CompilerParams(
            dimension_semantics=("parallel", "arbitrary")),
    )(adj, support, b.reshape(1, D_out))

    return out


# speedup vs baseline: 1.0170x; 1.0170x over previous
import functools

import jax
import jax.numpy as jnp
from jax import lax
from jax.experimental import pallas as pl
from jax.experimental.pallas import tpu as pltpu


def _support_body(x_ref, w_ref, out_ref):
    out_ref[...] = lax.dot_general(
        x_ref[...], w_ref[...], (((1,), (0,)), ((), ())),
        preferred_element_type=jnp.float32).astype(jnp.bfloat16)


def _agg_body(nk, bk, k_valid, adj_ref, sup_ref, b_ref, out_ref):
    k = pl.program_id(1)

    @pl.when(k == 0)
    def _init():
        out_ref[...] = jnp.zeros_like(out_ref)

    a = adj_ref[...].astype(jnp.bfloat16)
    col = k * bk + lax.broadcasted_iota(jnp.int32, a.shape, 1)
    a = jnp.where(col < k_valid, a, jnp.bfloat16(0.0))
    sup = sup_ref[pl.ds(k * bk, bk), :]
    out_ref[...] += lax.dot_general(
        a, sup, (((1,), (0,)), ((), ())),
        preferred_element_type=jnp.float32)

    @pl.when(k == nk - 1)
    def _finish():
        out_ref[...] = jnp.maximum(out_ref[...] + b_ref[...], 0.0)


@jax.jit
def kernel(x, adj, W, b):
    M, K = adj.shape
    D_in = x.shape[1]
    D_out = W.shape[1]

    BK = min(2048, pl.cdiv(K, 128) * 128)
    nk = pl.cdiv(K, BK)
    KP = nk * BK

    x_pad = jnp.pad(x, ((0, KP - K), (0, 0))) if KP != K else x

    support = pl.pallas_call(
        _support_body,
        grid=(KP // BK,),
        in_specs=[
            pl.BlockSpec((BK, D_in), lambda i: (i, 0)),
            pl.BlockSpec((D_in, D_out), lambda i: (0, 0)),
        ],
        out_specs=pl.BlockSpec((BK, D_out), lambda i: (i, 0)),
        out_shape=jax.ShapeDtypeStruct((KP, D_out), jnp.bfloat16),
    )(x_pad, W)

    BM = 400 if M % 400 == 0 else min(M, 256)
    nm = pl.cdiv(M, BM)

    out = pl.pallas_call(
        functools.partial(_agg_body, nk, BK, K),
        grid=(nm, nk),
        in_specs=[
            pl.BlockSpec((BM, BK), lambda i, k: (i, k)),
            pl.BlockSpec((KP, D_out), lambda i, k: (0, 0)),
            pl.BlockSpec((1, D_out), lambda i, k: (0, 0)),
        ],
        out_specs=pl.BlockSpec((BM, D_out), lambda i, k: (i, 0)),
        out_shape=jax.ShapeDtypeStruct((M, D_out), jnp.float32),
        compiler_params=pltpu.CompilerParams(
            dimension_semantics=("parallel", "arbitrary")),
    )(adj, support, b.reshape(1, D_out))

    return out


# speedup vs baseline: 1.4261x; 1.4023x over previous
import jax
import jax.numpy as jnp
from jax import lax
from jax.experimental import pallas as pl
from jax.experimental.pallas import tpu as pltpu


def _support_body(x_ref, w_ref, out_ref):
    out_ref[...] = lax.dot_general(
        x_ref[...], w_ref[...], (((1,), (0,)), ((), ())),
        preferred_element_type=jnp.float32).astype(jnp.bfloat16)


def _agg_body(adj_ref, sup_ref, b_ref, out_ref):
    a = adj_ref[...].astype(jnp.bfloat16)
    acc = lax.dot_general(
        a, sup_ref[...], (((1,), (0,)), ((), ())),
        preferred_element_type=jnp.float32)
    out_ref[...] = jnp.maximum(acc + b_ref[...], 0.0)


@jax.jit
def kernel(x, adj, W, b):
    M, K = adj.shape
    D_in = x.shape[1]
    D_out = W.shape[1]

    BK1 = K // 5 if K % 5 == 0 and (K // 5) % 8 == 0 else K
    support = pl.pallas_call(
        _support_body,
        grid=(K // BK1,),
        in_specs=[
            pl.BlockSpec((BK1, D_in), lambda i: (i, 0)),
            pl.BlockSpec((D_in, D_out), lambda i: (0, 0)),
        ],
        out_specs=pl.BlockSpec((BK1, D_out), lambda i: (i, 0)),
        out_shape=jax.ShapeDtypeStruct((K, D_out), jnp.bfloat16),
    )(x, W)

    BM = 400 if M % 400 == 0 else min(M, 256)
    nm = pl.cdiv(M, BM)

    out = pl.pallas_call(
        _agg_body,
        grid=(nm,),
        in_specs=[
            pl.BlockSpec((BM, K), lambda i: (i, 0)),
            pl.BlockSpec((K, D_out), lambda i: (0, 0)),
            pl.BlockSpec((1, D_out), lambda i: (0, 0)),
        ],
        out_specs=pl.BlockSpec((BM, D_out), lambda i: (i, 0)),
        out_shape=jax.ShapeDtypeStruct((M, D_out), jnp.float32),
        compiler_params=pltpu.CompilerParams(
            dimension_semantics=("parallel",)),
    )(adj, support, b.reshape(1, D_out))

    return out


# speedup vs baseline: 1.4738x; 1.0334x over previous
import jax
import jax.numpy as jnp
from jax import lax
from jax.experimental import pallas as pl
from jax.experimental.pallas import tpu as pltpu


def _fused_body(x_ref, w_ref, b_ref, adj_ref, out_ref, sup_ref):
    @pl.when(pl.program_id(0) == 0)
    def _support():
        xb = x_ref[...].astype(jnp.bfloat16)
        wb = w_ref[...].astype(jnp.bfloat16)
        sup_ref[...] = lax.dot_general(
            xb, wb, (((1,), (0,)), ((), ())),
            preferred_element_type=jnp.float32).astype(jnp.bfloat16)

    a = adj_ref[...].astype(jnp.bfloat16)
    acc = lax.dot_general(
        a, sup_ref[...], (((1,), (0,)), ((), ())),
        preferred_element_type=jnp.float32)
    out_ref[...] = jnp.maximum(acc + b_ref[...], 0.0)


@jax.jit
def kernel(x, adj, W, b):
    M, K = adj.shape
    D_in = x.shape[1]
    D_out = W.shape[1]

    BM = 200 if M % 200 == 0 else min(M, 256)
    nm = pl.cdiv(M, BM)

    out = pl.pallas_call(
        _fused_body,
        grid=(nm,),
        in_specs=[
            pl.BlockSpec((K, D_in), lambda i: (0, 0)),
            pl.BlockSpec((D_in, D_out), lambda i: (0, 0)),
            pl.BlockSpec((1, D_out), lambda i: (0, 0)),
            pl.BlockSpec((BM, K), lambda i: (i, 0)),
        ],
        out_specs=pl.BlockSpec((BM, D_out), lambda i: (i, 0)),
        out_shape=jax.ShapeDtypeStruct((M, D_out), jnp.float32),
        scratch_shapes=[pltpu.VMEM((K, D_out), jnp.bfloat16)],
        compiler_params=pltpu.CompilerParams(
            dimension_semantics=("arbitrary",)),
    )(x, W, b.reshape(1, D_out), adj)

    return out


# speedup vs baseline: 1.5003x; 1.0179x over previous
import jax
import jax.numpy as jnp
from jax import lax
from jax.experimental import pallas as pl
from jax.experimental.pallas import tpu as pltpu


def _fused_body(x_ref, w_ref, b_ref, adj_ref, out_ref, sup_ref):
    @pl.when(pl.program_id(0) == 0)
    def _support():
        xb = x_ref[...].astype(jnp.bfloat16)
        wb = w_ref[...].astype(jnp.bfloat16)
        sup_ref[...] = lax.dot_general(
            xb, wb, (((1,), (0,)), ((), ())),
            preferred_element_type=jnp.float32).astype(jnp.bfloat16)

    a = adj_ref[...].astype(jnp.bfloat16)
    acc = lax.dot_general(
        a, sup_ref[...], (((1,), (0,)), ((), ())),
        preferred_element_type=jnp.float32)
    out_ref[...] = jnp.maximum(acc + b_ref[...], 0.0)


@jax.jit
def kernel(x, adj, W, b):
    M, K = adj.shape
    D_in = x.shape[1]
    D_out = W.shape[1]

    BM = 400 if M % 400 == 0 else min(M, 256)
    nm = pl.cdiv(M, BM)

    out = pl.pallas_call(
        _fused_body,
        grid=(nm,),
        in_specs=[
            pl.BlockSpec((K, D_in), lambda i: (0, 0)),
            pl.BlockSpec((D_in, D_out), lambda i: (0, 0)),
            pl.BlockSpec((1, D_out), lambda i: (0, 0)),
            pl.BlockSpec((BM, K), lambda i: (i, 0)),
        ],
        out_specs=pl.BlockSpec((BM, D_out), lambda i: (i, 0)),
        out_shape=jax.ShapeDtypeStruct((M, D_out), jnp.float32),
        scratch_shapes=[pltpu.VMEM((K, D_out), jnp.bfloat16)],
        compiler_params=pltpu.CompilerParams(
            dimension_semantics=("arbitrary",)),
    )(x, W, b.reshape(1, D_out), adj)

    return out
